# baseline (speedup 1.0000x reference)
import jax
import jax.numpy as jnp
from jax.experimental import pallas as pl
from jax.experimental.pallas import tpu as pltpu

_WSLAB = {"w2": (0, 40, 80), "w3a": (40, 16, 40), "w3b": (56, 16, 40),
          "l1": (72, 32, 16), "l2": (104, 32, 32), "l3": (136, 32, 32),
          "l4": (168, 1, 32)}
_BSLAB = {"bhl": (0, 256), "bp": (256, 40), "b1": (296, 80), "b2": (376, 40),
          "b3": (416, 16), "l1b": (432, 32), "l2b": (464, 32),
          "l3b": (496, 32), "l4b": (528, 1)}


def _fused_gate_weights(rw):
    W0 = rw[0:32, 0:16]
    W1 = rw[32:64, 0:24]
    z = jnp.zeros((32, 8), rw.dtype)
    r0 = jnp.concatenate([W0[:, 0:8], W0[:, 8:16], z, z], axis=1)
    r1 = jnp.concatenate([z, W1[:, 8:16], W1[:, 0:8], W1[:, 16:24]], axis=1)
    W01 = jnp.concatenate([r0, r1], axis=0)
    perm = jnp.concatenate([jnp.arange(0, 24), jnp.arange(32, 56),
                            jnp.arange(24, 32), jnp.arange(56, 64)])
    Wp = W01[perm, :]
    scale = jnp.concatenate([jnp.full((48, 1), 0.5, Wp.dtype),
                             jnp.ones((16, 1), Wp.dtype)])
    return (Wp * scale).astype(jnp.bfloat16)


def _make_kernel(n_chunks, T):
    def body(x2_ref, wg_ref, wa_ref, w1_ref, wc_ref, tb_ref, out_ref,
             xs_ref, hbuf_ref):
        B = xs_ref.shape[2]
        C = B // n_chunks

        xs_ref[:, 0, :] = x2_ref[:, 0:T].T
        xs_ref[:, 1, :] = x2_ref[:, T:2 * T].T
        xs_ref[:, 2, :] = jnp.ones((T, B), jnp.float32)
        xs_ref[:, 3:8, :] = jnp.zeros((T, 5, B), jnp.float32)

        def dot(a, b):
            return jnp.dot(a, b, preferred_element_type=jnp.float32)

        def w(name):
            r0, nr, nc = _WSLAB[name]
            return wc_ref[r0:r0 + nr, 0:nc]

        def b(name):
            r0, n = _BSLAB[name]
            return tb_ref[r0:r0 + n, :]

        Wg = wg_ref[...]

        def step(t, carry):
            tx = jnp.minimum(t, T - 1)
            tm1 = jnp.maximum(t - 1, 0)
            x8 = xs_ref[tx]
            nxt = []
            for j in range(n_chunks):
                h1, c1, h2, c2, xp = carry[j]
                xj = x8[:, j * C:(j + 1) * C]
                s = jnp.concatenate([xj, h1, xp, h2], axis=0)
                g = dot(Wg, s.astype(jnp.bfloat16))
                sg = jnp.tanh(g[0:48, :]) * 0.5 + 0.5
                gt = jnp.tanh(g[48:64, :])
                c1 = sg[8:16, :] * c1 + sg[0:8, :] * gt[0:8, :]
                h1 = sg[16:24, :] * jnp.tanh(c1)
                c2 = sg[32:40, :] * c2 + sg[24:32, :] * gt[8:16, :]
                h2 = sg[40:48, :] * jnp.tanh(c2)
                hbuf_ref[tm1, :, j * C:(j + 1) * C] = h2
                nxt.append((h1, c1, h2, c2, xj))
            return tuple(nxt)

        z8 = jnp.zeros((8, C), jnp.float32)
        carry = tuple((z8, z8, z8, z8, z8) for _ in range(n_chunks))
        jax.lax.fori_loop(0, T + 1, step, carry, unroll=8)

        hflat = hbuf_ref[...].reshape(8 * T, B).astype(jnp.bfloat16)
        xflat = xs_ref[...].reshape(8 * T, B).astype(jnp.bfloat16)

        other = dot(wa_ref[256:296, :], xflat) + b("bp")
        hx = jnp.tanh(dot(wa_ref[0:256, :], hflat) + b("bhl"))
        z = jnp.tanh(dot(w1_ref[...], hx) + b("b1"))
        z = jnp.tanh(dot(w("w2"), z) + b("b2"))
        z = jax.nn.relu(dot(w("w3a"), z) + dot(w("w3b"), other) + b("b3"))
        z = jax.nn.relu(dot(w("l1"), z) + b("l1b"))
        z = jax.nn.relu(dot(w("l2"), z) + b("l2b"))
        z = jax.nn.relu(dot(w("l3"), z) + b("l3b"))
        out_ref[...] = jax.nn.sigmoid(dot(w("l4"), z) + b("l4b"))

    return body


def _lane_tile(Bp):
    for bt in (4096, 2048, 1024, 512, 256, 128):
        if Bp % bt == 0 and (Bp // bt >= 2 or bt == 128):
            return bt
    return Bp


def kernel(x, rw, wA, w1, wC, tb):
    B, _, T = x.shape
    x = x.astype(jnp.float32)

    Bp = ((B + 127) // 128) * 128
    bt = _lane_tile(Bp)
    n_chunks = 1
    grid = (Bp // bt,)

    x2 = x.reshape(B, 2 * T)
    if Bp != B:
        x2 = jnp.pad(x2, ((0, Bp - B), (0, 0)))
    wg = _fused_gate_weights(rw)
    wA = wA.astype(jnp.bfloat16)

    def whole(a):
        nd = a.ndim
        return pl.BlockSpec(a.shape, lambda i, _n=nd: (0,) * _n)

    out = pl.pallas_call(
        _make_kernel(n_chunks, T),
        out_shape=jax.ShapeDtypeStruct((1, Bp), jnp.float32),
        grid=grid,
        in_specs=[pl.BlockSpec((bt, 2 * T), lambda i: (i, 0)),
                  whole(wg), whole(wA), whole(w1), whole(wC), whole(tb)],
        out_specs=pl.BlockSpec((1, bt), lambda i: (0, i)),
        scratch_shapes=[pltpu.VMEM((T, 8, bt), jnp.float32),
                        pltpu.VMEM((T, 8, bt), jnp.float32)],
        compiler_params=pltpu.CompilerParams(dimension_semantics=("parallel",)),
    )(x2, wg, wA, w1, wC, tb)

    return out[:, :B].T


# speedup vs baseline: 1.0761x; 1.0761x over previous
import jax
import jax.numpy as jnp
from jax.experimental import pallas as pl
from jax.experimental.pallas import tpu as pltpu

_WSLAB = {"w2": (0, 40, 80), "w3a": (40, 16, 40), "w3b": (56, 16, 40),
          "l1": (72, 32, 16), "l2": (104, 32, 32), "l3": (136, 32, 32),
          "l4": (168, 1, 32)}
_BSLAB = {"bhl": (0, 256), "bp": (256, 40), "b1": (296, 80), "b2": (376, 40),
          "b3": (416, 16), "l1b": (432, 32), "l2b": (464, 32),
          "l3b": (496, 32), "l4b": (528, 1)}


def _fused_gate_weights(rw):
    W0 = rw[0:32, 0:16]
    W1 = rw[32:64, 0:24]
    z = jnp.zeros((32, 8), rw.dtype)
    r0 = jnp.concatenate([W0[:, 0:8], W0[:, 8:16], z, z], axis=1)
    r1 = jnp.concatenate([z, W1[:, 8:16], W1[:, 0:8], W1[:, 16:24]], axis=1)
    W01 = jnp.concatenate([r0, r1], axis=0)
    perm = jnp.concatenate([jnp.arange(0, 24), jnp.arange(32, 56),
                            jnp.arange(24, 32), jnp.arange(56, 64)])
    Wp = W01[perm, :]
    scale = jnp.concatenate([jnp.full((48, 1), 0.5, Wp.dtype),
                             jnp.ones((16, 1), Wp.dtype)])
    return Wp * scale


def _make_kernel(n_chunks, T):
    def body(x2_ref, wg_ref, wa_ref, w1_ref, wc_ref, tb_ref, out_ref,
             xs_ref, hbuf_ref):
        B = xs_ref.shape[2]
        C = B // n_chunks

        xs_ref[:, 0, :] = x2_ref[:, 0:T].T
        xs_ref[:, 1, :] = x2_ref[:, T:2 * T].T
        xs_ref[:, 2, :] = jnp.ones((T, B), jnp.float32)
        xs_ref[:, 3:8, :] = jnp.zeros((T, 5, B), jnp.float32)

        def dot(a, b):
            return jnp.dot(a, b, preferred_element_type=jnp.float32)

        def w(name):
            r0, nr, nc = _WSLAB[name]
            return wc_ref[r0:r0 + nr, 0:nc]

        def b(name):
            r0, n = _BSLAB[name]
            return tb_ref[r0:r0 + n, :]

        Wg = wg_ref[...]

        def step(t, carry):
            tx = jnp.minimum(t, T - 1)
            tm1 = jnp.maximum(t - 1, 0)
            x8 = xs_ref[tx]
            nxt = []
            for j in range(n_chunks):
                h1, c1, h2, c2, xp = carry[j]
                xj = x8[:, j * C:(j + 1) * C]
                s = jnp.concatenate([xj, h1, xp, h2], axis=0)
                g = dot(Wg, s)
                sg = jnp.tanh(g[0:48, :]) * 0.5 + 0.5
                gt = jnp.tanh(g[48:64, :])
                c1 = sg[8:16, :] * c1 + sg[0:8, :] * gt[0:8, :]
                h1 = sg[16:24, :] * jnp.tanh(c1)
                c2 = sg[32:40, :] * c2 + sg[24:32, :] * gt[8:16, :]
                h2 = sg[40:48, :] * jnp.tanh(c2)
                hbuf_ref[tm1, :, j * C:(j + 1) * C] = h2
                nxt.append((h1, c1, h2, c2, xj))
            return tuple(nxt)

        z8 = jnp.zeros((8, C), jnp.float32)
        carry = tuple((z8, z8, z8, z8, z8) for _ in range(n_chunks))
        jax.lax.fori_loop(0, T + 1, step, carry, unroll=8)

        hflat = hbuf_ref[...].reshape(8 * T, B)
        xflat = xs_ref[...].reshape(8 * T, B)

        other = dot(wa_ref[256:296, :], xflat) + b("bp")
        hx = jnp.tanh(dot(wa_ref[0:256, :], hflat) + b("bhl"))
        z = jnp.tanh(dot(w1_ref[...], hx) + b("b1"))
        z = jnp.tanh(dot(w("w2"), z) + b("b2"))
        z = jax.nn.relu(dot(w("w3a"), z) + dot(w("w3b"), other) + b("b3"))
        z = jax.nn.relu(dot(w("l1"), z) + b("l1b"))
        z = jax.nn.relu(dot(w("l2"), z) + b("l2b"))
        z = jax.nn.relu(dot(w("l3"), z) + b("l3b"))
        out_ref[...] = jax.nn.sigmoid(dot(w("l4"), z) + b("l4b"))

    return body


def _lane_tile(Bp):
    for bt in (4096, 2048, 1024, 512, 256, 128):
        if Bp % bt == 0 and (Bp // bt >= 2 or bt == 128):
            return bt
    return Bp


def kernel(x, rw, wA, w1, wC, tb):
    B, _, T = x.shape
    x = x.astype(jnp.float32)

    Bp = ((B + 127) // 128) * 128
    bt = _lane_tile(Bp)
    n_chunks = 1
    grid = (Bp // bt,)

    x2 = x.reshape(B, 2 * T)
    if Bp != B:
        x2 = jnp.pad(x2, ((0, Bp - B), (0, 0)))
    wg = _fused_gate_weights(rw)

    def whole(a):
        nd = a.ndim
        return pl.BlockSpec(a.shape, lambda i, _n=nd: (0,) * _n)

    out = pl.pallas_call(
        _make_kernel(n_chunks, T),
        out_shape=jax.ShapeDtypeStruct((1, Bp), jnp.float32),
        grid=grid,
        in_specs=[pl.BlockSpec((bt, 2 * T), lambda i: (i, 0)),
                  whole(wg), whole(wA), whole(w1), whole(wC), whole(tb)],
        out_specs=pl.BlockSpec((1, bt), lambda i: (0, i)),
        scratch_shapes=[pltpu.VMEM((T, 8, bt), jnp.float32),
                        pltpu.VMEM((T, 8, bt), jnp.float32)],
        compiler_params=pltpu.CompilerParams(dimension_semantics=("parallel",)),
    )(x2, wg, wA, w1, wC, tb)

    return out[:, :B].T


# speedup vs baseline: 1.0769x; 1.0007x over previous
import jax
import jax.numpy as jnp
from jax.experimental import pallas as pl
from jax.experimental.pallas import tpu as pltpu

_WSLAB = {"w2": (0, 40, 80), "w3a": (40, 16, 40), "w3b": (56, 16, 40),
          "l1": (72, 32, 16), "l2": (104, 32, 32), "l3": (136, 32, 32),
          "l4": (168, 1, 32)}
_BSLAB = {"bhl": (0, 256), "bp": (256, 40), "b1": (296, 80), "b2": (376, 40),
          "b3": (416, 16), "l1b": (432, 32), "l2b": (464, 32),
          "l3b": (496, 32), "l4b": (528, 1)}


def _fused_gate_weights(rw):
    W0 = rw[0:32, 0:16]
    W1 = rw[32:64, 0:24]
    z = jnp.zeros((32, 8), rw.dtype)
    r0 = jnp.concatenate([W0[:, 0:8], W0[:, 8:16], z, z], axis=1)
    r1 = jnp.concatenate([z, W1[:, 8:16], W1[:, 0:8], W1[:, 16:24]], axis=1)
    W01 = jnp.concatenate([r0, r1], axis=0)
    perm = jnp.concatenate([jnp.arange(0, 24), jnp.arange(32, 56),
                            jnp.arange(24, 32), jnp.arange(56, 64)])
    Wp = W01[perm, :]
    scale = jnp.concatenate([jnp.full((48, 1), 0.5, Wp.dtype),
                             jnp.ones((16, 1), Wp.dtype)])
    return Wp * scale


def _make_kernel(n_chunks, T):
    def body(x2_ref, wg_ref, wa_ref, w1_ref, wc_ref, tb_ref, out_ref,
             xs_ref, hbuf_ref):
        B = xs_ref.shape[2]
        C = B // n_chunks

        xs_ref[:, 0, :] = x2_ref[:, 0:T].T
        xs_ref[:, 1, :] = x2_ref[:, T:2 * T].T
        xs_ref[:, 2, :] = jnp.ones((T, B), jnp.float32)
        xs_ref[:, 3:8, :] = jnp.zeros((T, 5, B), jnp.float32)

        def dot(a, b):
            return jnp.dot(a, b, preferred_element_type=jnp.float32)

        def w(name):
            r0, nr, nc = _WSLAB[name]
            return wc_ref[r0:r0 + nr, 0:nc]

        def b(name):
            r0, n = _BSLAB[name]
            return tb_ref[r0:r0 + n, :]

        Wg = wg_ref[...]

        def step(t, carry):
            tx = jnp.minimum(t, T - 1)
            tm1 = jnp.maximum(t - 1, 0)
            x8 = xs_ref[tx]
            nxt = []
            for j in range(n_chunks):
                h1, c1, h2, c2, xp = carry[j]
                xj = x8[:, j * C:(j + 1) * C]
                s = jnp.concatenate([xj, h1, xp, h2], axis=0)
                g = dot(Wg, s)
                sg = jnp.tanh(g[0:48, :]) * 0.5 + 0.5
                gt = jnp.tanh(g[48:64, :])
                c1 = sg[8:16, :] * c1 + sg[0:8, :] * gt[0:8, :]
                h1 = sg[16:24, :] * jnp.tanh(c1)
                c2 = sg[32:40, :] * c2 + sg[24:32, :] * gt[8:16, :]
                h2 = sg[40:48, :] * jnp.tanh(c2)
                hbuf_ref[tm1, :, j * C:(j + 1) * C] = h2
                nxt.append((h1, c1, h2, c2, xj))
            return tuple(nxt)

        z8 = jnp.zeros((8, C), jnp.float32)
        carry = tuple((z8, z8, z8, z8, z8) for _ in range(n_chunks))
        jax.lax.fori_loop(0, T + 1, step, carry, unroll=8)

        hflat = hbuf_ref[...].reshape(8 * T, B)
        xflat = xs_ref[...].reshape(8 * T, B)

        other = dot(wa_ref[256:296, :], xflat) + b("bp")
        hx = jnp.tanh(dot(wa_ref[0:256, :], hflat) + b("bhl"))
        z = jnp.tanh(dot(w1_ref[...], hx) + b("b1"))
        z = jnp.tanh(dot(w("w2"), z) + b("b2"))
        z = jax.nn.relu(dot(w("w3a"), z) + dot(w("w3b"), other) + b("b3"))
        z = jax.nn.relu(dot(w("l1"), z) + b("l1b"))
        z = jax.nn.relu(dot(w("l2"), z) + b("l2b"))
        z = jax.nn.relu(dot(w("l3"), z) + b("l3b"))
        out_ref[...] = jax.nn.sigmoid(dot(w("l4"), z) + b("l4b"))

    return body


def _lane_tile(Bp):
    for bt in (4096, 2048, 1024, 512, 256, 128):
        if Bp % bt == 0 and (Bp // bt >= 2 or bt == 128):
            return bt
    return Bp


def kernel(x, rw, wA, w1, wC, tb):
    B, _, T = x.shape
    x = x.astype(jnp.float32)

    Bp = ((B + 127) // 128) * 128
    bt = _lane_tile(Bp)
    n_chunks = 2 if bt >= 4096 else 1
    grid = (Bp // bt,)

    x2 = x.reshape(B, 2 * T)
    if Bp != B:
        x2 = jnp.pad(x2, ((0, Bp - B), (0, 0)))
    wg = _fused_gate_weights(rw)

    def whole(a):
        nd = a.ndim
        return pl.BlockSpec(a.shape, lambda i, _n=nd: (0,) * _n)

    out = pl.pallas_call(
        _make_kernel(n_chunks, T),
        out_shape=jax.ShapeDtypeStruct((1, Bp), jnp.float32),
        grid=grid,
        in_specs=[pl.BlockSpec((bt, 2 * T), lambda i: (i, 0)),
                  whole(wg), whole(wA), whole(w1), whole(wC), whole(tb)],
        out_specs=pl.BlockSpec((1, bt), lambda i: (0, i)),
        scratch_shapes=[pltpu.VMEM((T, 8, bt), jnp.float32),
                        pltpu.VMEM((T, 8, bt), jnp.float32)],
        compiler_params=pltpu.CompilerParams(dimension_semantics=("parallel",)),
    )(x2, wg, wA, w1, wC, tb)

    return out[:, :B].T


# speedup vs baseline: 1.0952x; 1.0170x over previous
import jax
import jax.numpy as jnp
from jax.experimental import pallas as pl
from jax.experimental.pallas import tpu as pltpu

_WSLAB = {"w2": (0, 40, 80), "w3a": (40, 16, 40), "w3b": (56, 16, 40),
          "l1": (72, 32, 16), "l2": (104, 32, 32), "l3": (136, 32, 32),
          "l4": (168, 1, 32)}
_BSLAB = {"bhl": (0, 256), "bp": (256, 40), "b1": (296, 80), "b2": (376, 40),
          "b3": (416, 16), "l1b": (432, 32), "l2b": (464, 32),
          "l3b": (496, 32), "l4b": (528, 1)}


def _fused_gate_weights(rw):
    W0 = rw[0:32, 0:16]
    W1 = rw[32:64, 0:24]
    z = jnp.zeros((32, 8), rw.dtype)
    r0 = jnp.concatenate([z, W0[:, 0:8], W0[:, 8:16], z], axis=1)
    r1 = jnp.concatenate([W1[:, 0:8], z, W1[:, 8:16], W1[:, 16:24]], axis=1)
    W01 = jnp.concatenate([r0, r1], axis=0)
    perm = jnp.concatenate([jnp.arange(0, 24), jnp.arange(32, 56),
                            jnp.arange(24, 32), jnp.arange(56, 64)])
    scale = jnp.concatenate([jnp.full((48, 1), 0.5, rw.dtype),
                             jnp.ones((16, 1), rw.dtype)])
    return W01[perm, :] * scale


def _passthrough_weights(wA, tb, T):
    wao = wA[256:296, :].reshape(40, T, 8)
    keep = wao * (jnp.arange(8) < 3).astype(wA.dtype)
    shifted = jnp.concatenate(
        [jnp.zeros((40, 1, 8), wA.dtype), keep,
         jnp.zeros((40, 1, 8), wA.dtype)], axis=1)
    return shifted.reshape(40, 8 * (T + 2))


def _make_kernel(T):
    def body(x2_ref, wg_ref, wa_ref, wao_ref, w1_ref, wc_ref, tb_ref,
             out_ref, xs_ref, hbuf_ref):
        B = xs_ref.shape[2]

        xs_ref[0] = jnp.zeros((8, B), jnp.float32)
        xs_ref[T + 1] = jnp.zeros((8, B), jnp.float32)
        xs_ref[1:T + 1, 3:8, :] = jnp.zeros((T, 5, B), jnp.float32)
        xs_ref[1:T + 1, 0, :] = x2_ref[:, 0:T].T
        xs_ref[1:T + 1, 1, :] = x2_ref[:, T:2 * T].T
        xs_ref[1:T + 1, 2, :] = jnp.ones((T, B), jnp.float32)

        def dot(a, b):
            return jnp.dot(a, b, preferred_element_type=jnp.float32)

        def w(name):
            r0, nr, nc = _WSLAB[name]
            return wc_ref[r0:r0 + nr, 0:nc]

        def b(name):
            r0, n = _BSLAB[name]
            return tb_ref[r0:r0 + n, :]

        Wg = wg_ref[...]

        def step(t, carry):
            h1, c1, h2, c2 = carry
            xw = xs_ref[pl.ds(t, 2)].reshape(16, B)
            s = jnp.concatenate([xw, h1, h2], axis=0)
            g = dot(Wg, s)
            sg = jnp.tanh(g[0:48, :]) * 0.5 + 0.5
            gt = jnp.tanh(g[48:64, :])
            c1 = sg[8:16, :] * c1 + sg[0:8, :] * gt[0:8, :]
            h1 = sg[16:24, :] * jnp.tanh(c1)
            c2 = sg[32:40, :] * c2 + sg[24:32, :] * gt[8:16, :]
            h2 = sg[40:48, :] * jnp.tanh(c2)
            hbuf_ref[jnp.maximum(t - 1, 0)] = h2
            return h1, c1, h2, c2

        z8 = jnp.zeros((8, B), jnp.float32)
        jax.lax.fori_loop(0, T + 1, step, (z8, z8, z8, z8), unroll=8)

        hflat = hbuf_ref[...].reshape(8 * T, B)
        xflat = xs_ref[...].reshape(8 * (T + 2), B)

        other = dot(wao_ref[...], xflat) + b("bp")
        hx = jnp.tanh(dot(wa_ref[0:256, :], hflat) + b("bhl"))
        z = jnp.tanh(dot(w1_ref[...], hx) + b("b1"))
        z = jnp.tanh(dot(w("w2"), z) + b("b2"))
        z = jax.nn.relu(dot(w("w3a"), z) + dot(w("w3b"), other) + b("b3"))
        z = jax.nn.relu(dot(w("l1"), z) + b("l1b"))
        z = jax.nn.relu(dot(w("l2"), z) + b("l2b"))
        z = jax.nn.relu(dot(w("l3"), z) + b("l3b"))
        out_ref[...] = jax.nn.sigmoid(dot(w("l4"), z) + b("l4b"))

    return body


def _lane_tile(Bp):
    for bt in (4096, 2048, 1024, 512, 256, 128):
        if Bp % bt == 0 and (Bp // bt >= 2 or bt == 128):
            return bt
    return Bp


def kernel(x, rw, wA, w1, wC, tb):
    B, _, T = x.shape
    x = x.astype(jnp.float32)

    Bp = ((B + 127) // 128) * 128
    bt = _lane_tile(Bp)
    grid = (Bp // bt,)

    x2 = x.reshape(B, 2 * T)
    if Bp != B:
        x2 = jnp.pad(x2, ((0, Bp - B), (0, 0)))
    wg = _fused_gate_weights(rw)
    wao = _passthrough_weights(wA, tb, T)

    def whole(a):
        nd = a.ndim
        return pl.BlockSpec(a.shape, lambda i, _n=nd: (0,) * _n)

    out = pl.pallas_call(
        _make_kernel(T),
        out_shape=jax.ShapeDtypeStruct((1, Bp), jnp.float32),
        grid=grid,
        in_specs=[pl.BlockSpec((bt, 2 * T), lambda i: (i, 0)),
                  whole(wg), whole(wA), whole(wao), whole(w1), whole(wC),
                  whole(tb)],
        out_specs=pl.BlockSpec((1, bt), lambda i: (0, i)),
        scratch_shapes=[pltpu.VMEM((T + 2, 8, bt), jnp.float32),
                        pltpu.VMEM((T, 8, bt), jnp.float32)],
        compiler_params=pltpu.CompilerParams(dimension_semantics=("parallel",)),
    )(x2, wg, wA, wao, w1, wC, tb)

    return out[:, :B].T


# speedup vs baseline: 1.1444x; 1.0449x over previous
import jax
import jax.numpy as jnp
from jax.experimental import pallas as pl
from jax.experimental.pallas import tpu as pltpu

_WSLAB = {"w2": (0, 40, 80), "w3a": (40, 16, 40), "w3b": (56, 16, 40),
          "l1": (72, 32, 16), "l2": (104, 32, 32), "l3": (136, 32, 32),
          "l4": (168, 1, 32)}
_BSLAB = {"bhl": (0, 256), "bp": (256, 40), "b1": (296, 80), "b2": (376, 40),
          "b3": (416, 16), "l1b": (432, 32), "l2b": (464, 32),
          "l3b": (496, 32), "l4b": (528, 1)}


def _fused_gate_weights(rw):
    W0 = rw[0:32, 0:16]
    W1 = rw[32:64, 0:24]
    z = jnp.zeros((32, 8), rw.dtype)
    r0 = jnp.concatenate([z, W0[:, 0:8], W0[:, 8:16], z], axis=1)
    r1 = jnp.concatenate([W1[:, 0:8], z, W1[:, 8:16], W1[:, 16:24]], axis=1)
    W01 = jnp.concatenate([r0, r1], axis=0)
    perm = jnp.concatenate([jnp.arange(0, 24), jnp.arange(32, 56),
                            jnp.arange(24, 32), jnp.arange(56, 64)])
    scale = jnp.concatenate([jnp.full((48, 1), 0.5, rw.dtype),
                             jnp.ones((16, 1), rw.dtype)])
    return W01[perm, :] * scale


def _passthrough_weights(wA, tb, T):
    wao = wA[256:296, :].reshape(40, T, 8)
    keep = wao * (jnp.arange(8) < 3).astype(wA.dtype)
    shifted = jnp.concatenate(
        [jnp.zeros((40, 1, 8), wA.dtype), keep,
         jnp.zeros((40, 1, 8), wA.dtype)], axis=1)
    return shifted.reshape(40, 8 * (T + 2))


def _make_kernel(T):
    def body(x2_ref, wg_ref, wa_ref, wao_ref, w1_ref, wc_ref, tb_ref,
             out_ref, xs_ref, hbuf_ref):
        B = xs_ref.shape[2]

        xs_ref[0] = jnp.zeros((8, B), jnp.float32)
        xs_ref[T + 1] = jnp.zeros((8, B), jnp.float32)
        xs_ref[1:T + 1, 3:8, :] = jnp.zeros((T, 5, B), jnp.float32)
        xs_ref[1:T + 1, 0, :] = x2_ref[:, 0:T].T
        xs_ref[1:T + 1, 1, :] = x2_ref[:, T:2 * T].T
        xs_ref[1:T + 1, 2, :] = jnp.ones((T, B), jnp.float32)

        def dot(a, b):
            return jnp.dot(a, b, preferred_element_type=jnp.float32)

        def w(name):
            r0, nr, nc = _WSLAB[name]
            return wc_ref[r0:r0 + nr, 0:nc]

        def b(name):
            r0, n = _BSLAB[name]
            return tb_ref[r0:r0 + n, :]

        Wg = wg_ref[...]

        def step(t, carry):
            h1, c1, h2, c2 = carry
            xw = xs_ref[pl.ds(t, 2)].reshape(16, B)
            s = jnp.concatenate([xw, h1, h2], axis=0)
            g = dot(Wg, s)
            sg = jnp.tanh(g[0:48, :]) * 0.5 + 0.5
            gt = jnp.tanh(g[48:64, :])
            c1 = sg[8:16, :] * c1 + sg[0:8, :] * gt[0:8, :]
            h1 = sg[16:24, :] * jnp.tanh(c1)
            c2 = sg[32:40, :] * c2 + sg[24:32, :] * gt[8:16, :]
            h2 = sg[40:48, :] * jnp.tanh(c2)
            hbuf_ref[jnp.maximum(t - 1, 0)] = h2
            return h1, c1, h2, c2

        z8 = jnp.zeros((8, B), jnp.float32)
        jax.lax.fori_loop(0, T + 1, step, (z8, z8, z8, z8), unroll=16)

        hflat = hbuf_ref[...].reshape(8 * T, B)
        xflat = xs_ref[...].reshape(8 * (T + 2), B)

        other = dot(wao_ref[...], xflat) + b("bp")
        hx = jnp.tanh(dot(wa_ref[0:256, :], hflat) + b("bhl"))
        z = jnp.tanh(dot(w1_ref[...], hx) + b("b1"))
        z = jnp.tanh(dot(w("w2"), z) + b("b2"))
        z = jax.nn.relu(dot(w("w3a"), z) + dot(w("w3b"), other) + b("b3"))
        z = jax.nn.relu(dot(w("l1"), z) + b("l1b"))
        z = jax.nn.relu(dot(w("l2"), z) + b("l2b"))
        z = jax.nn.relu(dot(w("l3"), z) + b("l3b"))
        out_ref[...] = jax.nn.sigmoid(dot(w("l4"), z) + b("l4b"))

    return body


def _lane_tile(Bp):
    for bt in (4096, 2048, 1024, 512, 256, 128):
        if Bp % bt == 0 and (Bp // bt >= 2 or bt == 128):
            return bt
    return Bp


def kernel(x, rw, wA, w1, wC, tb):
    B, _, T = x.shape
    x = x.astype(jnp.float32)

    Bp = ((B + 127) // 128) * 128
    bt = _lane_tile(Bp)
    grid = (Bp // bt,)

    x2 = x.reshape(B, 2 * T)
    if Bp != B:
        x2 = jnp.pad(x2, ((0, Bp - B), (0, 0)))
    wg = _fused_gate_weights(rw)
    wao = _passthrough_weights(wA, tb, T)

    def whole(a):
        nd = a.ndim
        return pl.BlockSpec(a.shape, lambda i, _n=nd: (0,) * _n)

    out = pl.pallas_call(
        _make_kernel(T),
        out_shape=jax.ShapeDtypeStruct((1, Bp), jnp.float32),
        grid=grid,
        in_specs=[pl.BlockSpec((bt, 2 * T), lambda i: (i, 0)),
                  whole(wg), whole(wA), whole(wao), whole(w1), whole(wC),
                  whole(tb)],
        out_specs=pl.BlockSpec((1, bt), lambda i: (0, i)),
        scratch_shapes=[pltpu.VMEM((T + 2, 8, bt), jnp.float32),
                        pltpu.VMEM((T, 8, bt), jnp.float32)],
        compiler_params=pltpu.CompilerParams(dimension_semantics=("parallel",)),
    )(x2, wg, wA, wao, w1, wC, tb)

    return out[:, :B].T


# speedup vs baseline: 1.1829x; 1.0336x over previous
import jax
import jax.numpy as jnp
from jax.experimental import pallas as pl
from jax.experimental.pallas import tpu as pltpu

_WSLAB = {"w2": (0, 40, 80), "w3a": (40, 16, 40), "w3b": (56, 16, 40),
          "l1": (72, 32, 16), "l2": (104, 32, 32), "l3": (136, 32, 32),
          "l4": (168, 1, 32)}
_BSLAB = {"bhl": (0, 256), "bp": (256, 40), "b1": (296, 80), "b2": (376, 40),
          "b3": (416, 16), "l1b": (432, 32), "l2b": (464, 32),
          "l3b": (496, 32), "l4b": (528, 1)}


def _fused_gate_weights(rw):
    W0 = rw[0:32, 0:16]
    W1 = rw[32:64, 0:24]
    z = jnp.zeros((32, 8), rw.dtype)
    r0 = jnp.concatenate([z, W0[:, 0:8], W0[:, 8:16], z], axis=1)
    r1 = jnp.concatenate([W1[:, 0:8], z, W1[:, 8:16], W1[:, 16:24]], axis=1)
    W01 = jnp.concatenate([r0, r1], axis=0)
    perm = jnp.concatenate([jnp.arange(0, 24), jnp.arange(32, 56),
                            jnp.arange(24, 32), jnp.arange(56, 64)])
    scale = jnp.concatenate([jnp.full((48, 1), 0.5, rw.dtype),
                             jnp.ones((16, 1), rw.dtype)])
    return W01[perm, :] * scale


def _passthrough_weights(wA, tb, T):
    wao = wA[256:296, :].reshape(40, T, 8)
    keep = wao * (jnp.arange(8) < 3).astype(wA.dtype)
    shifted = jnp.concatenate(
        [jnp.zeros((40, 1, 8), wA.dtype), keep,
         jnp.zeros((40, 1, 8), wA.dtype)], axis=1)
    return shifted.reshape(40, 8 * (T + 2))


def _make_kernel(T):
    def body(x2_ref, wg_ref, wa_ref, wao_ref, w1_ref, wc_ref, tb_ref,
             out_ref, xs_ref, hbuf_ref):
        B = xs_ref.shape[2]

        xs_ref[0] = jnp.zeros((8, B), jnp.float32)
        xs_ref[T + 1] = jnp.zeros((8, B), jnp.float32)
        xs_ref[1:T + 1, 3:8, :] = jnp.zeros((T, 5, B), jnp.float32)
        xs_ref[1:T + 1, 0, :] = x2_ref[:, 0:T].T
        xs_ref[1:T + 1, 1, :] = x2_ref[:, T:2 * T].T
        xs_ref[1:T + 1, 2, :] = jnp.ones((T, B), jnp.float32)

        def dot(a, b):
            return jnp.dot(a, b, preferred_element_type=jnp.float32)

        def w(name):
            r0, nr, nc = _WSLAB[name]
            return wc_ref[r0:r0 + nr, 0:nc]

        def b(name):
            r0, n = _BSLAB[name]
            return tb_ref[r0:r0 + n, :]

        Wg = wg_ref[...]

        def step(t, carry):
            h1, c1, h2, c2 = carry
            xw = xs_ref[pl.ds(t, 2)].reshape(16, B)
            s = jnp.concatenate([xw, h1, h2], axis=0)
            g = dot(Wg, s)
            sg = jnp.tanh(g[0:48, :]) * 0.5 + 0.5
            gt = jnp.tanh(g[48:64, :])
            c1 = sg[8:16, :] * c1 + sg[0:8, :] * gt[0:8, :]
            h1 = sg[16:24, :] * jnp.tanh(c1)
            c2 = sg[32:40, :] * c2 + sg[24:32, :] * gt[8:16, :]
            h2 = sg[40:48, :] * jnp.tanh(c2)
            hbuf_ref[jnp.maximum(t - 1, 0)] = h2
            return h1, c1, h2, c2

        z8 = jnp.zeros((8, B), jnp.float32)
        jax.lax.fori_loop(0, T + 1, step, (z8, z8, z8, z8), unroll=64)

        hflat = hbuf_ref[...].reshape(8 * T, B)
        xflat = xs_ref[...].reshape(8 * (T + 2), B)

        other = dot(wao_ref[...], xflat) + b("bp")
        hx = jnp.tanh(dot(wa_ref[0:256, :], hflat) + b("bhl"))
        z = jnp.tanh(dot(w1_ref[...], hx) + b("b1"))
        z = jnp.tanh(dot(w("w2"), z) + b("b2"))
        z = jax.nn.relu(dot(w("w3a"), z) + dot(w("w3b"), other) + b("b3"))
        z = jax.nn.relu(dot(w("l1"), z) + b("l1b"))
        z = jax.nn.relu(dot(w("l2"), z) + b("l2b"))
        z = jax.nn.relu(dot(w("l3"), z) + b("l3b"))
        out_ref[...] = jax.nn.sigmoid(dot(w("l4"), z) + b("l4b"))

    return body


def _lane_tile(Bp):
    for bt in (4096, 2048, 1024, 512, 256, 128):
        if Bp % bt == 0 and (Bp // bt >= 2 or bt == 128):
            return bt
    return Bp


def kernel(x, rw, wA, w1, wC, tb):
    B, _, T = x.shape
    x = x.astype(jnp.float32)

    Bp = ((B + 127) // 128) * 128
    bt = _lane_tile(Bp)
    grid = (Bp // bt,)

    x2 = x.reshape(B, 2 * T)
    if Bp != B:
        x2 = jnp.pad(x2, ((0, Bp - B), (0, 0)))
    wg = _fused_gate_weights(rw)
    wao = _passthrough_weights(wA, tb, T)

    def whole(a):
        nd = a.ndim
        return pl.BlockSpec(a.shape, lambda i, _n=nd: (0,) * _n)

    out = pl.pallas_call(
        _make_kernel(T),
        out_shape=jax.ShapeDtypeStruct((1, Bp), jnp.float32),
        grid=grid,
        in_specs=[pl.BlockSpec((bt, 2 * T), lambda i: (i, 0)),
                  whole(wg), whole(wA), whole(wao), whole(w1), whole(wC),
                  whole(tb)],
        out_specs=pl.BlockSpec((1, bt), lambda i: (0, i)),
        scratch_shapes=[pltpu.VMEM((T + 2, 8, bt), jnp.float32),
                        pltpu.VMEM((T, 8, bt), jnp.float32)],
        compiler_params=pltpu.CompilerParams(dimension_semantics=("parallel",)),
    )(x2, wg, wA, wao, w1, wC, tb)

    return out[:, :B].T
